# Initial kernel scaffold; baseline (speedup 1.0000x reference)
import functools

import jax
import jax.numpy as jnp
from jax import lax
from jax.experimental import pallas as pl
from jax.experimental.pallas import tpu as pltpu, tpu_sc as plsc

N = 10000
D = 128
DOUT = 40
E = 320000

NC = 2
NS = 16
NW = NC * NS
CH = 128
NPAD = 10240
RPT = NPAD // NS
TRASH = 10016
CHUNKS = -(-E // (NW * CH))
TOT_E = NW * CH * CHUNKS
PAD_E = TOT_E - E
BR = 1024

_mesh = plsc.VectorSubcoreMesh(core_axis_name="c", subcore_axis_name="s")


@functools.partial(
    pl.kernel,
    out_type=jax.ShapeDtypeStruct((NC * NPAD,), jnp.float32),
    mesh=_mesh,
    scratch_types=[
        pltpu.VMEM((CH,), jnp.int32),
        pltpu.VMEM((CH,), jnp.float32),
        pltpu.VMEM((RPT,), jnp.float32),
        pltpu.VMEM_SHARED((NPAD,), jnp.float32),
    ],
)
def _deg_kernel(dst_hbm, out_hbm, idx_v, ones_v, zero_v, deg_s):
    cid = lax.axis_index("c")
    sid = lax.axis_index("s")
    wid = cid * NS + sid
    rbase = sid * RPT
    for i in range(CH // 16):
        ones_v[pl.ds(i * 16, 16)] = jnp.full((16,), 1.0, jnp.float32)
    for i in range(RPT // 16):
        zero_v[pl.ds(i * 16, 16)] = jnp.zeros((16,), jnp.float32)
    pltpu.sync_copy(zero_v, deg_s.at[pl.ds(rbase, RPT)])
    plsc.subcore_barrier()

    def body(c, carry):
        base = (wid * CHUNKS + c) * CH
        pltpu.sync_copy(dst_hbm.at[pl.ds(base, CH)], idx_v)
        pltpu.sync_copy(ones_v, deg_s.at[idx_v], add=True)
        return carry

    lax.fori_loop(0, CHUNKS, body, 0)
    plsc.subcore_barrier()
    pltpu.sync_copy(deg_s.at[pl.ds(rbase, RPT)],
                    out_hbm.at[pl.ds(cid * NPAD + rbase, RPT)])


@functools.partial(
    pl.kernel,
    out_type=jax.ShapeDtypeStruct((NC * NPAD, D), jnp.float32),
    mesh=_mesh,
    scratch_types=[
        pltpu.VMEM((CH,), jnp.int32),
        pltpu.VMEM((CH,), jnp.int32),
        pltpu.VMEM((CH, D), jnp.float32),
        pltpu.VMEM_SHARED((NPAD, D), jnp.float32),
        pltpu.SemaphoreType.DMA,
    ],
)
def _scatter_kernel(g_hbm, src_hbm, dst_hbm, zsrc_hbm, out_hbm,
                    sidx, didx, rows, acc_s, sem):
    cid = lax.axis_index("c")
    sid = lax.axis_index("s")
    wid = cid * NS + sid
    rbase = sid * RPT

    @pl.when(cid == 0)
    def _():
        pltpu.sync_copy(g_hbm.at[pl.ds(rbase, RPT)], acc_s.at[pl.ds(rbase, RPT)])

    @pl.when(cid != 0)
    def _():
        pltpu.sync_copy(zsrc_hbm, acc_s.at[pl.ds(rbase, RPT)])

    plsc.subcore_barrier()

    def body(c, carry):
        base = (wid * CHUNKS + c) * CH
        pltpu.sync_copy(src_hbm.at[pl.ds(base, CH)], sidx)
        pltpu.sync_copy(dst_hbm.at[pl.ds(base, CH)], didx)
        pltpu.async_copy(g_hbm.at[sidx], rows, sem).wait()
        pltpu.sync_copy(rows, acc_s.at[didx], add=True)
        return carry

    lax.fori_loop(0, CHUNKS, body, 0)
    plsc.subcore_barrier()
    pltpu.sync_copy(acc_s.at[pl.ds(rbase, RPT)],
                    out_hbm.at[pl.ds(cid * NPAD + rbase, RPT)])


def _prep_body(x_ref, d0_ref, d1_ref, g_ref, dinv_ref, dginv_ref):
    deg = d0_ref[...] + d1_ref[...] + 1.0
    dv = lax.rsqrt(deg)
    g_ref[...] = x_ref[...] * dv
    dinv_ref[...] = dv
    dginv_ref[...] = dv * dv


_prep = pl.pallas_call(
    _prep_body,
    grid=(NPAD // BR,),
    in_specs=[
        pl.BlockSpec((BR, D), lambda i: (i, 0)),
        pl.BlockSpec((BR, 1), lambda i: (i, 0)),
        pl.BlockSpec((BR, 1), lambda i: (i, 0)),
    ],
    out_specs=[
        pl.BlockSpec((BR, D), lambda i: (i, 0)),
        pl.BlockSpec((BR, 1), lambda i: (i, 0)),
        pl.BlockSpec((BR, 1), lambda i: (i, 0)),
    ],
    out_shape=[
        jax.ShapeDtypeStruct((NPAD, D), jnp.float32),
        jax.ShapeDtypeStruct((NPAD, 1), jnp.float32),
        jax.ShapeDtypeStruct((NPAD, 1), jnp.float32),
    ],
)


def _combine_body(s0_ref, s1_ref, dginv_ref, g_ref):
    g_ref[...] = (s0_ref[...] + s1_ref[...]) * dginv_ref[...]


_combine = pl.pallas_call(
    _combine_body,
    grid=(NPAD // BR,),
    in_specs=[
        pl.BlockSpec((BR, D), lambda i: (i, 0)),
        pl.BlockSpec((BR, D), lambda i: (i, 0)),
        pl.BlockSpec((BR, 1), lambda i: (i, 0)),
    ],
    out_specs=pl.BlockSpec((BR, D), lambda i: (i, 0)),
    out_shape=jax.ShapeDtypeStruct((NPAD, D), jnp.float32),
)


def _final_body(s0_ref, s1_ref, dinv_ref, wt_ref, b_ref, out_ref, emb_ref):
    h2 = (s0_ref[...] + s1_ref[...]) * dinv_ref[...]
    emb = jnp.dot(h2, wt_ref[...], preferred_element_type=jnp.float32) + b_ref[...]
    col = lax.broadcasted_iota(jnp.int32, emb.shape, 1)
    logits = jnp.where(col < DOUT, emb, -1e30)
    m = jnp.max(logits, axis=1, keepdims=True)
    lse = jnp.log(jnp.sum(jnp.exp(logits - m), axis=1, keepdims=True)) + m
    emb_ref[...] = emb
    out_ref[...] = emb - lse


_final = pl.pallas_call(
    _final_body,
    grid=(NPAD // BR,),
    in_specs=[
        pl.BlockSpec((BR, D), lambda i: (i, 0)),
        pl.BlockSpec((BR, D), lambda i: (i, 0)),
        pl.BlockSpec((BR, 1), lambda i: (i, 0)),
        pl.BlockSpec((D, D), lambda i: (0, 0)),
        pl.BlockSpec((1, D), lambda i: (0, 0)),
    ],
    out_specs=[
        pl.BlockSpec((BR, D), lambda i: (i, 0)),
        pl.BlockSpec((BR, D), lambda i: (i, 0)),
    ],
    out_shape=[
        jax.ShapeDtypeStruct((NPAD, D), jnp.float32),
        jax.ShapeDtypeStruct((NPAD, D), jnp.float32),
    ],
)


@jax.jit
def kernel(x, edge_index, W, b):
    src_p = jnp.concatenate([edge_index[0], jnp.zeros((PAD_E,), jnp.int32)])
    dst_p = jnp.concatenate([edge_index[1], jnp.full((PAD_E,), TRASH, jnp.int32)])
    x_p = jnp.pad(x, ((0, NPAD - N), (0, 0)))
    wt = jnp.zeros((D, D), jnp.float32).at[:, :DOUT].set(W.T)
    bp = jnp.zeros((1, D), jnp.float32).at[0, :DOUT].set(b)
    zsrc = jnp.zeros((RPT, D), jnp.float32)

    degp = _deg_kernel(dst_p)
    g0, dinv, dginv = _prep(x_p, degp[:NPAD, None], degp[NPAD:, None])
    s1 = _scatter_kernel(g0, src_p, dst_p, zsrc)
    g1 = _combine(s1[:NPAD], s1[NPAD:], dginv)
    s2 = _scatter_kernel(g1, src_p, dst_p, zsrc)
    outp, embp = _final(s2[:NPAD], s2[NPAD:], dinv, wt, bp)
    return outp[:N, :DOUT], embp[:N, :DOUT]


# speedup vs baseline: 10.0092x; 10.0092x over previous
import functools

import jax
import jax.numpy as jnp
from jax import lax
from jax.experimental import pallas as pl
from jax.experimental.pallas import tpu as pltpu, tpu_sc as plsc

N = 10000
D = 128
DOUT = 40
E = 320000

NC = 2
NS = 16
NW = NC * NS
CH = 128
NPAD = 10240
RPT = NPAD // NS
TRASH = 10016
CHUNKS = -(-E // (NW * CH))
TOT_E = NW * CH * CHUNKS
PAD_E = TOT_E - E
BR = 1024

_mesh = plsc.VectorSubcoreMesh(
    core_axis_name="c", subcore_axis_name="s", num_cores=NC, num_subcores=NS)


@functools.partial(
    pl.kernel,
    out_type=jax.ShapeDtypeStruct((NC * NPAD,), jnp.float32),
    mesh=_mesh,
    scratch_types=[
        pltpu.VMEM((CH,), jnp.int32),
        pltpu.VMEM((CH,), jnp.float32),
        pltpu.VMEM((RPT,), jnp.float32),
        pltpu.VMEM_SHARED((NPAD,), jnp.float32),
    ],
)
def _deg_kernel(dst_hbm, out_hbm, idx_v, ones_v, zero_v, deg_s):
    cid = lax.axis_index("c")
    sid = lax.axis_index("s")
    wid = cid * NS + sid
    rbase = sid * RPT
    for i in range(CH // 16):
        ones_v[pl.ds(i * 16, 16)] = jnp.full((16,), 1.0, jnp.float32)
    for i in range(RPT // 16):
        zero_v[pl.ds(i * 16, 16)] = jnp.zeros((16,), jnp.float32)
    pltpu.sync_copy(zero_v, deg_s.at[pl.ds(rbase, RPT)])
    plsc.subcore_barrier()

    def body(c, carry):
        base = (wid * CHUNKS + c) * CH
        pltpu.sync_copy(dst_hbm.at[pl.ds(base, CH)], idx_v)
        pltpu.sync_copy(ones_v, deg_s.at[idx_v], add=True)
        return carry

    lax.fori_loop(0, CHUNKS, body, 0)
    plsc.subcore_barrier()
    pltpu.sync_copy(deg_s.at[pl.ds(rbase, RPT)],
                    out_hbm.at[pl.ds(cid * NPAD + rbase, RPT)])


@functools.partial(
    pl.kernel,
    out_type=jax.ShapeDtypeStruct((NC * NPAD, D), jnp.float32),
    mesh=_mesh,
    scratch_types=[
        pltpu.VMEM((CH,), jnp.int32),
        pltpu.VMEM((CH,), jnp.int32),
        pltpu.VMEM((CH, D), jnp.float32),
        pltpu.VMEM_SHARED((NPAD, D), jnp.float32),
        pltpu.SemaphoreType.DMA,
    ],
)
def _scatter_kernel(g_hbm, src_hbm, dst_hbm, zsrc_hbm, out_hbm,
                    sidx, didx, rows, acc_s, sem):
    cid = lax.axis_index("c")
    sid = lax.axis_index("s")
    wid = cid * NS + sid
    rbase = sid * RPT

    @pl.when(cid == 0)
    def _():
        pltpu.sync_copy(g_hbm.at[pl.ds(rbase, RPT)], acc_s.at[pl.ds(rbase, RPT)])

    @pl.when(cid != 0)
    def _():
        pltpu.sync_copy(zsrc_hbm, acc_s.at[pl.ds(rbase, RPT)])

    plsc.subcore_barrier()

    def body(c, carry):
        base = (wid * CHUNKS + c) * CH
        pltpu.sync_copy(src_hbm.at[pl.ds(base, CH)], sidx)
        pltpu.sync_copy(dst_hbm.at[pl.ds(base, CH)], didx)
        pltpu.async_copy(g_hbm.at[sidx], rows, sem).wait()
        pltpu.sync_copy(rows, acc_s.at[didx], add=True)
        return carry

    lax.fori_loop(0, CHUNKS, body, 0)
    plsc.subcore_barrier()
    pltpu.sync_copy(acc_s.at[pl.ds(rbase, RPT)],
                    out_hbm.at[pl.ds(cid * NPAD + rbase, RPT)])


def _prep_body(x_ref, d0_ref, d1_ref, g_ref, dinv_ref, dginv_ref):
    deg = d0_ref[...] + d1_ref[...] + 1.0
    dv = lax.rsqrt(deg)
    g_ref[...] = x_ref[...] * dv
    dinv_ref[...] = dv
    dginv_ref[...] = dv * dv


_prep = pl.pallas_call(
    _prep_body,
    grid=(NPAD // BR,),
    in_specs=[
        pl.BlockSpec((BR, D), lambda i: (i, 0)),
        pl.BlockSpec((BR, 1), lambda i: (i, 0)),
        pl.BlockSpec((BR, 1), lambda i: (i, 0)),
    ],
    out_specs=[
        pl.BlockSpec((BR, D), lambda i: (i, 0)),
        pl.BlockSpec((BR, 1), lambda i: (i, 0)),
        pl.BlockSpec((BR, 1), lambda i: (i, 0)),
    ],
    out_shape=[
        jax.ShapeDtypeStruct((NPAD, D), jnp.float32),
        jax.ShapeDtypeStruct((NPAD, 1), jnp.float32),
        jax.ShapeDtypeStruct((NPAD, 1), jnp.float32),
    ],
)


def _combine_body(s0_ref, s1_ref, dginv_ref, g_ref):
    g_ref[...] = (s0_ref[...] + s1_ref[...]) * dginv_ref[...]


_combine = pl.pallas_call(
    _combine_body,
    grid=(NPAD // BR,),
    in_specs=[
        pl.BlockSpec((BR, D), lambda i: (i, 0)),
        pl.BlockSpec((BR, D), lambda i: (i, 0)),
        pl.BlockSpec((BR, 1), lambda i: (i, 0)),
    ],
    out_specs=pl.BlockSpec((BR, D), lambda i: (i, 0)),
    out_shape=jax.ShapeDtypeStruct((NPAD, D), jnp.float32),
)


def _final_body(s0_ref, s1_ref, dinv_ref, wt_ref, b_ref, out_ref, emb_ref):
    h2 = (s0_ref[...] + s1_ref[...]) * dinv_ref[...]
    emb = jnp.dot(h2, wt_ref[...], preferred_element_type=jnp.float32) + b_ref[...]
    col = lax.broadcasted_iota(jnp.int32, emb.shape, 1)
    logits = jnp.where(col < DOUT, emb, -1e30)
    m = jnp.max(logits, axis=1, keepdims=True)
    lse = jnp.log(jnp.sum(jnp.exp(logits - m), axis=1, keepdims=True)) + m
    emb_ref[...] = emb
    out_ref[...] = emb - lse


_final = pl.pallas_call(
    _final_body,
    grid=(NPAD // BR,),
    in_specs=[
        pl.BlockSpec((BR, D), lambda i: (i, 0)),
        pl.BlockSpec((BR, D), lambda i: (i, 0)),
        pl.BlockSpec((BR, 1), lambda i: (i, 0)),
        pl.BlockSpec((D, D), lambda i: (0, 0)),
        pl.BlockSpec((1, D), lambda i: (0, 0)),
    ],
    out_specs=[
        pl.BlockSpec((BR, D), lambda i: (i, 0)),
        pl.BlockSpec((BR, D), lambda i: (i, 0)),
    ],
    out_shape=[
        jax.ShapeDtypeStruct((NPAD, D), jnp.float32),
        jax.ShapeDtypeStruct((NPAD, D), jnp.float32),
    ],
)


@jax.jit
def kernel(x, edge_index, W, b):
    src_p = jnp.concatenate([edge_index[0], jnp.zeros((PAD_E,), jnp.int32)])
    dst_p = jnp.concatenate([edge_index[1], jnp.full((PAD_E,), TRASH, jnp.int32)])
    x_p = jnp.pad(x, ((0, NPAD - N), (0, 0)))
    wt = jnp.zeros((D, D), jnp.float32).at[:, :DOUT].set(W.T)
    bp = jnp.zeros((1, D), jnp.float32).at[0, :DOUT].set(b)
    zsrc = jnp.zeros((RPT, D), jnp.float32)

    degp = _deg_kernel(dst_p)
    g0, dinv, dginv = _prep(x_p, degp[:NPAD, None], degp[NPAD:, None])
    s1 = _scatter_kernel(g0, src_p, dst_p, zsrc)
    g1 = _combine(s1[:NPAD], s1[NPAD:], dginv)
    s2 = _scatter_kernel(g1, src_p, dst_p, zsrc)
    outp, embp = _final(s2[:NPAD], s2[NPAD:], dinv, wt, bp)
    return outp[:N, :DOUT], embp[:N, :DOUT]
